# speedup vs baseline: 10.0387x; 10.0387x over previous
import jax
import jax.numpy as jnp
from jax import lax
from jax.experimental import pallas as pl
from jax.experimental.pallas import tpu as pltpu
from jax.experimental.pallas import tpu_sc as plsc

B, N, D, C = 4, 65536, 128, 512
NC, NS, L = 2, 16, 16
GRP = NS // 2
PTS = N // GRP
CHUNK = 128
NCH = PTS // CHUNK
CPB = C // GRP
RPT = (2 * C) // NS


def _body(emb_hbm, tgt_hbm, out_hbm,
          stage0, stage1, tgt_ref, ones_ref, pbuf, cbuf,
          acc_proto, acc_cnt, sem0, sem1):
    core = lax.axis_index("c")
    s = lax.axis_index("s")
    g = s // GRP
    cb = s % GRP
    b = 2 * core + g

    zf = jnp.zeros((L,), jnp.float32)

    def zrow(r, c):
        for k in range(D // L):
            stage0[r, pl.ds(k * L, L)] = zf
        cbuf[r, pl.ds(0, L)] = zf
        return c
    lax.fori_loop(0, RPT, zrow, 0)

    def orow(r, c):
        ones_ref[r, pl.ds(0, L)] = jnp.ones((L,), jnp.float32)
        return c
    lax.fori_loop(0, CHUNK, orow, 0)

    pltpu.sync_copy(stage0.at[pl.ds(0, RPT)], acc_proto.at[pl.ds(s * RPT, RPT)])
    pltpu.sync_copy(cbuf, acc_cnt.at[pl.ds(s * RPT, RPT)])

    pltpu.sync_copy(tgt_hbm.at[b, pl.ds(cb * NCH, NCH)], tgt_ref)
    bias = jnp.full((L,), g * C, jnp.int32)

    def brow(r, c):
        for k in range(CHUNK // L):
            tgt_ref[r, pl.ds(k * L, L)] = tgt_ref[r, pl.ds(k * L, L)] + bias
        return c
    lax.fori_loop(0, NCH, brow, 0)

    plsc.subcore_barrier()

    base = cb * PTS
    stages = (stage0, stage1)
    sems = (sem0, sem1)

    pltpu.async_copy(emb_hbm.at[b, pl.ds(base, CHUNK)], stage0, sem0)
    pltpu.async_copy(emb_hbm.at[b, pl.ds(base + CHUNK, CHUNK)], stage1, sem1)

    def chunk_body(gi, c):
        for p in range(2):
            i = 2 * gi + p
            pltpu.make_async_copy(
                emb_hbm.at[b, pl.ds(base + i * CHUNK, CHUNK)],
                stages[p], sems[p]).wait()
            pltpu.sync_copy(stages[p], acc_proto.at[tgt_ref.at[i]], add=True)
            pltpu.sync_copy(ones_ref, acc_cnt.at[tgt_ref.at[i]], add=True)

            @pl.when(i + 2 < NCH)
            def _():
                pltpu.async_copy(
                    emb_hbm.at[b, pl.ds(base + (i + 2) * CHUNK, CHUNK)],
                    stages[p], sems[p])
        return c
    lax.fori_loop(0, NCH // 2, chunk_body, 0)

    plsc.subcore_barrier()

    row0 = g * C + cb * CPB
    pltpu.sync_copy(acc_proto.at[pl.ds(row0, CPB)], pbuf)
    pltpu.sync_copy(acc_cnt.at[pl.ds(row0, CPB)], cbuf)

    def drow(r, c):
        cnt = cbuf[r, pl.ds(0, L)]
        recip = 1.0 / jnp.maximum(cnt, 1.0)
        for k in range(D // L):
            pbuf[r, pl.ds(k * L, L)] = pbuf[r, pl.ds(k * L, L)] * recip
        return c
    lax.fori_loop(0, CPB, drow, 0)

    pltpu.sync_copy(pbuf, out_hbm.at[b, pl.ds(cb * CPB, CPB)])


def kernel(embeddings, targets):
    tgt = targets.astype(jnp.int32).reshape(B, N // CHUNK, CHUNK)
    f = pl.kernel(
        _body,
        out_type=jax.ShapeDtypeStruct((B, C, D), jnp.float32),
        mesh=plsc.VectorSubcoreMesh(core_axis_name="c", subcore_axis_name="s"),
        scratch_types=[
            pltpu.VMEM((CHUNK, D), jnp.float32),
            pltpu.VMEM((CHUNK, D), jnp.float32),
            pltpu.VMEM((NCH, CHUNK), jnp.int32),
            pltpu.VMEM((CHUNK, L), jnp.float32),
            pltpu.VMEM((CPB, D), jnp.float32),
            pltpu.VMEM((CPB, L), jnp.float32),
            pltpu.VMEM_SHARED((2 * C, D), jnp.float32),
            pltpu.VMEM_SHARED((2 * C, L), jnp.float32),
            pltpu.SemaphoreType.DMA,
            pltpu.SemaphoreType.DMA,
        ],
    )
    return f(embeddings, tgt)
